# baseline (speedup 1.0000x reference)
import functools

import jax
import jax.numpy as jnp
from jax import lax
from jax.experimental import pallas as pl
from jax.experimental.pallas import tpu as pltpu
from jax.experimental.pallas import tpu_sc as plsc

N = 100000
D = 128
GRID = 5
BLK = N // GRID

_NT = (((1,), (1,)), ((), ()))

NSUB = 16
CHUNK = 6400
LAST_LEN = N - (NSUB - 1) * CHUNK
NEG = float("-inf")
BIG = 2**31 - 1


def _cos_kernel(e_ref, col_ref, cos_ref):
    e = e_ref[...]
    qn = e / jnp.sqrt(jnp.sum(e * e) + 1e-12)
    x = col_ref[...]
    ones = jnp.ones((1, D), jnp.float32)
    ss = lax.dot_general(ones, x * x, _NT, precision=lax.Precision.HIGHEST,
                         preferred_element_type=jnp.float32)
    dp = lax.dot_general(qn, x, _NT, precision=lax.Precision.HIGHEST,
                         preferred_element_type=jnp.float32)
    cos_ref[...] = (dp / jnp.sqrt(ss + 1e-12))[None]


def _lanes():
    return lax.iota(jnp.int32, 16)


def _permute(x, perm):
    return lax.gather(
        x, perm[:, None],
        lax.GatherDimensionNumbers(offset_dims=(), collapsed_slice_dims=(0,),
                                   start_index_map=(0,)),
        slice_sizes=(1,),
        mode=lax.GatherScatterMode.PROMISE_IN_BOUNDS)


def _bfly_argmax(m, pi):
    lanes = _lanes()
    for k in (1, 2, 4, 8):
        perm = jnp.bitwise_xor(lanes, k)
        gm = _permute(m, perm)
        gp = _permute(pi, perm)
        better = (gm > m) | ((gm == m) & (gp < pi))
        m = jnp.where(better, gm, m)
        pi = jnp.where(better, gp, pi)
    return m, pi


def _bfly_max(v):
    lanes = _lanes()
    for k in (1, 2, 4, 8):
        v = jnp.maximum(v, _permute(v, jnp.bitwise_xor(lanes, k)))
    return v


def _bfly_min(v):
    lanes = _lanes()
    for k in (1, 2, 4, 8):
        v = jnp.minimum(v, _permute(v, jnp.bitwise_xor(lanes, k)))
    return v


def _sc_vote(cos_hbm, lab_hbm, pred_hbm, conf_hbm, nconf_hbm,
             cos_v, vec_f, vec_i, cand_v, cpos_v, idx_v, lab_v,
             sh_vals, sh_pos, sem):
    core = lax.axis_index("c")
    sub = lax.axis_index("s")
    lanes = _lanes()
    lane0 = lanes == 0

    @pl.when(core == 0)
    def _worker():
        base = sub * CHUNK

        @pl.when(sub == NSUB - 1)
        def _pad_tail():
            def fill(i, _):
                cos_v[pl.ds(LAST_LEN + i * 16, 16)] = (
                    jnp.full((16,), NEG, jnp.float32))
                return 0
            lax.fori_loop(0, (CHUNK - LAST_LEN) // 16, fill, 0)

        pltpu.sync_copy(cos_hbm.at[pl.ds(base, LAST_LEN)],
                        cos_v.at[pl.ds(0, LAST_LEN)])

        @pl.when(sub < NSUB - 1)
        def _rest():
            pltpu.sync_copy(cos_hbm.at[pl.ds(base + LAST_LEN,
                                             CHUNK - LAST_LEN)],
                            cos_v.at[pl.ds(LAST_LEN, CHUNK - LAST_LEN)])

        acc_v = jnp.full((16,), NEG, jnp.float32)
        acc_p = jnp.zeros((16,), jnp.int32)
        for t in range(10):
            def body(i, carry):
                m, pi, idx = carry
                v = cos_v[pl.ds(i * 16, 16)]
                gt = v > m
                return (jnp.where(gt, v, m), jnp.where(gt, idx, pi),
                        idx + 16)
            m, pi, _ = lax.fori_loop(
                0, CHUNK // 16, body,
                (jnp.full((16,), NEG, jnp.float32),
                 jnp.zeros((16,), jnp.int32), lanes))
            mx, pmin = _bfly_argmax(m, pi)
            acc_v = jnp.where(lanes == t, mx, acc_v)
            acc_p = jnp.where(lanes == t, pmin, acc_p)
            plsc.store_scatter(cos_v, [pmin],
                               jnp.full((16,), NEG, jnp.float32), mask=lane0)
        vec_f[...] = acc_v
        vec_i[...] = acc_p
        pltpu.touch(vec_f)
        pltpu.touch(vec_i)
        pl.delay(200)
        pltpu.sync_copy(vec_f, sh_vals.at[pl.ds(sub * 16, 16)])
        pltpu.sync_copy(vec_i, sh_pos.at[pl.ds(sub * 16, 16)])

    plsc.subcore_barrier()

    @pl.when((core == 0) & (sub == 0))
    def _merge():
        for w in range(NSUB):
            pltpu.sync_copy(sh_vals.at[pl.ds(w * 16, 16)],
                            cand_v.at[pl.ds(w * 16, 16)])
            pltpu.sync_copy(sh_pos.at[pl.ds(w * 16, 16)],
                            cpos_v.at[pl.ds(w * 16, 16)])
        nb_v = jnp.full((16,), NEG, jnp.float32)
        nb_g = jnp.zeros((16,), jnp.int32)
        for t in range(10):
            m = jnp.full((16,), NEG, jnp.float32)
            pi = jnp.zeros((16,), jnp.int32)
            for w in range(NSUB):
                v = cand_v[pl.ds(w * 16, 16)]
                gt = v > m
                m = jnp.where(gt, v, m)
                pi = jnp.where(gt, lanes + (w * 16), pi)
            mx, pmin = _bfly_argmax(m, pi)
            lp = plsc.load_gather(cpos_v, [pmin])
            gpos = lp + (pmin // 16) * CHUNK
            if t >= 1:
                nb_v = jnp.where(lanes == t - 1, mx, nb_v)
                nb_g = jnp.where(lanes == t - 1, gpos, nb_g)
            plsc.store_scatter(cand_v, [pmin],
                               jnp.full((16,), NEG, jnp.float32), mask=lane0)
        idx_v[...] = nb_g
        pltpu.async_copy(lab_hbm.at[idx_v], lab_v, sem).wait()
        lv = lab_v[...]
        valid = lanes < 9
        cnts = jnp.zeros((16,), jnp.int32)
        for k in range(9):
            bk = _permute(lv, jnp.full((16,), k, jnp.int32))
            cnts = cnts + jnp.where(lv == bk, 1, 0)
        cnts = jnp.where(valid, cnts, 0)
        best = _bfly_max(cnts)
        winner = _bfly_min(jnp.where((cnts == best) & valid, lv, BIG))
        firstj = _bfly_min(jnp.where((lv == winner) & valid, lanes, BIG))
        conf = _bfly_max(jnp.where(lanes == firstj, nb_v, NEG))
        nconf = best.astype(jnp.float32) / jnp.float32(9.0)
        vec_i[...] = winner
        pltpu.touch(vec_i)
        pl.delay(200)
        pltpu.sync_copy(vec_i, pred_hbm)
        vec_f[...] = conf
        pltpu.touch(vec_f)
        pl.delay(200)
        pltpu.sync_copy(vec_f, conf_hbm)
        idx_v[...] = jnp.zeros((16,), jnp.int32)
        vec_f[...] = nconf
        pltpu.touch(vec_f)
        pl.delay(200)
        pltpu.sync_copy(vec_f, nconf_hbm)


_vote_call = functools.partial(
    pl.kernel,
    out_type=[
        jax.ShapeDtypeStruct((16,), jnp.int32),
        jax.ShapeDtypeStruct((16,), jnp.float32),
        jax.ShapeDtypeStruct((16,), jnp.float32),
    ],
    mesh=plsc.VectorSubcoreMesh(core_axis_name="c", subcore_axis_name="s",
                                num_cores=2, num_subcores=NSUB),
    compiler_params=pltpu.CompilerParams(needs_layout_passes=False),
    scratch_types=[
        pltpu.VMEM((CHUNK,), jnp.float32),
        pltpu.VMEM((16,), jnp.float32),
        pltpu.VMEM((16,), jnp.int32),
        pltpu.VMEM((NSUB * 16,), jnp.float32),
        pltpu.VMEM((NSUB * 16,), jnp.int32),
        pltpu.VMEM((16,), jnp.int32),
        pltpu.VMEM((16,), jnp.int32),
        pltpu.VMEM_SHARED((NSUB * 16,), jnp.float32),
        pltpu.VMEM_SHARED((NSUB * 16,), jnp.int32),
        pltpu.SemaphoreType.DMA,
    ],
)(_sc_vote)


def kernel(embedding, embedding_collection, labels_int):
    cos = pl.pallas_call(
        _cos_kernel,
        grid=(GRID,),
        in_specs=[
            pl.BlockSpec((1, D), lambda i: (0, 0)),
            pl.BlockSpec((BLK, D), lambda i: (i, 0)),
        ],
        out_specs=pl.BlockSpec((1, 1, BLK), lambda i: (i, 0, 0)),
        out_shape=jax.ShapeDtypeStruct((GRID, 1, BLK), jnp.float32),
    )(embedding, embedding_collection)
    pred, conf, nconf = _vote_call(cos.reshape(N), labels_int)
    return (pred[0], conf[0], nconf[0])


# speedup vs baseline: 1.9030x; 1.9030x over previous
import functools

import jax
import jax.numpy as jnp
from jax import lax
from jax.experimental import pallas as pl
from jax.experimental.pallas import tpu as pltpu
from jax.experimental.pallas import tpu_sc as plsc

N = 100000
D = 128
GRID = 5
BLK = N // GRID

_NT = (((1,), (1,)), ((), ()))

NSUB = 16
CHUNK = 6400
LAST_LEN = N - (NSUB - 1) * CHUNK
NEG = float("-inf")
BIG = 2**31 - 1


def _cos_kernel(e_ref, col_ref, cos_ref):
    e = e_ref[...]
    qn = e / jnp.sqrt(jnp.sum(e * e) + 1e-12)
    x = col_ref[...]
    ones = jnp.ones((1, D), jnp.float32)
    ss = lax.dot_general(ones, x * x, _NT,
                         preferred_element_type=jnp.float32)
    dp = lax.dot_general(qn, x, _NT,
                         preferred_element_type=jnp.float32)
    cos_ref[...] = (dp / jnp.sqrt(ss + 1e-12))[None]


def _lanes():
    return lax.iota(jnp.int32, 16)


def _permute(x, perm):
    return lax.gather(
        x, perm[:, None],
        lax.GatherDimensionNumbers(offset_dims=(), collapsed_slice_dims=(0,),
                                   start_index_map=(0,)),
        slice_sizes=(1,),
        mode=lax.GatherScatterMode.PROMISE_IN_BOUNDS)


def _bfly_argmax(m, pi):
    lanes = _lanes()
    for k in (1, 2, 4, 8):
        perm = jnp.bitwise_xor(lanes, k)
        gm = _permute(m, perm)
        gp = _permute(pi, perm)
        better = (gm > m) | ((gm == m) & (gp < pi))
        m = jnp.where(better, gm, m)
        pi = jnp.where(better, gp, pi)
    return m, pi


def _bfly_max(v):
    lanes = _lanes()
    for k in (1, 2, 4, 8):
        v = jnp.maximum(v, _permute(v, jnp.bitwise_xor(lanes, k)))
    return v


def _bfly_min(v):
    lanes = _lanes()
    for k in (1, 2, 4, 8):
        v = jnp.minimum(v, _permute(v, jnp.bitwise_xor(lanes, k)))
    return v


def _sc_vote(cos_hbm, lab_hbm, pred_hbm, conf_hbm, nconf_hbm,
             cos_v, vec_f, vec_i, cand_v, cpos_v, idx_v, lab_v,
             sh_vals, sh_pos, sem):
    core = lax.axis_index("c")
    sub = lax.axis_index("s")
    lanes = _lanes()
    lane0 = lanes == 0

    @pl.when(core == 0)
    def _worker():
        base = sub * CHUNK

        @pl.when(sub == NSUB - 1)
        def _pad_tail():
            def fill(i, _):
                cos_v[pl.ds(LAST_LEN + i * 16, 16)] = (
                    jnp.full((16,), NEG, jnp.float32))
                return 0
            lax.fori_loop(0, (CHUNK - LAST_LEN) // 16, fill, 0)

        pltpu.sync_copy(cos_hbm.at[pl.ds(base, LAST_LEN)],
                        cos_v.at[pl.ds(0, LAST_LEN)])

        @pl.when(sub < NSUB - 1)
        def _rest():
            pltpu.sync_copy(cos_hbm.at[pl.ds(base + LAST_LEN,
                                             CHUNK - LAST_LEN)],
                            cos_v.at[pl.ds(LAST_LEN, CHUNK - LAST_LEN)])

        acc_v = jnp.full((16,), NEG, jnp.float32)
        acc_p = jnp.zeros((16,), jnp.int32)
        for t in range(10):
            def body(i, carry):
                m, pi, idx = carry
                v = cos_v[pl.ds(i * 16, 16)]
                gt = v > m
                return (jnp.where(gt, v, m), jnp.where(gt, idx, pi),
                        idx + 16)
            m, pi, _ = lax.fori_loop(
                0, CHUNK // 16, body,
                (jnp.full((16,), NEG, jnp.float32),
                 jnp.zeros((16,), jnp.int32), lanes))
            mx, pmin = _bfly_argmax(m, pi)
            acc_v = jnp.where(lanes == t, mx, acc_v)
            acc_p = jnp.where(lanes == t, pmin, acc_p)
            plsc.store_scatter(cos_v, [pmin],
                               jnp.full((16,), NEG, jnp.float32), mask=lane0)
        vec_f[...] = acc_v
        vec_i[...] = acc_p
        pltpu.touch(vec_f)
        pltpu.touch(vec_i)
        pl.delay(200)
        pltpu.sync_copy(vec_f, sh_vals.at[pl.ds(sub * 16, 16)])
        pltpu.sync_copy(vec_i, sh_pos.at[pl.ds(sub * 16, 16)])

    plsc.subcore_barrier()

    @pl.when((core == 0) & (sub == 0))
    def _merge():
        for w in range(NSUB):
            pltpu.sync_copy(sh_vals.at[pl.ds(w * 16, 16)],
                            cand_v.at[pl.ds(w * 16, 16)])
            pltpu.sync_copy(sh_pos.at[pl.ds(w * 16, 16)],
                            cpos_v.at[pl.ds(w * 16, 16)])
        nb_v = jnp.full((16,), NEG, jnp.float32)
        nb_g = jnp.zeros((16,), jnp.int32)
        for t in range(10):
            m = jnp.full((16,), NEG, jnp.float32)
            pi = jnp.zeros((16,), jnp.int32)
            for w in range(NSUB):
                v = cand_v[pl.ds(w * 16, 16)]
                gt = v > m
                m = jnp.where(gt, v, m)
                pi = jnp.where(gt, lanes + (w * 16), pi)
            mx, pmin = _bfly_argmax(m, pi)
            lp = plsc.load_gather(cpos_v, [pmin])
            gpos = lp + (pmin // 16) * CHUNK
            if t >= 1:
                nb_v = jnp.where(lanes == t - 1, mx, nb_v)
                nb_g = jnp.where(lanes == t - 1, gpos, nb_g)
            plsc.store_scatter(cand_v, [pmin],
                               jnp.full((16,), NEG, jnp.float32), mask=lane0)
        idx_v[...] = nb_g
        pltpu.async_copy(lab_hbm.at[idx_v], lab_v, sem).wait()
        lv = lab_v[...]
        valid = lanes < 9
        cnts = jnp.zeros((16,), jnp.int32)
        for k in range(9):
            bk = _permute(lv, jnp.full((16,), k, jnp.int32))
            cnts = cnts + jnp.where(lv == bk, 1, 0)
        cnts = jnp.where(valid, cnts, 0)
        best = _bfly_max(cnts)
        winner = _bfly_min(jnp.where((cnts == best) & valid, lv, BIG))
        firstj = _bfly_min(jnp.where((lv == winner) & valid, lanes, BIG))
        conf = _bfly_max(jnp.where(lanes == firstj, nb_v, NEG))
        nconf = best.astype(jnp.float32) / jnp.float32(9.0)
        vec_i[...] = winner
        pltpu.touch(vec_i)
        pl.delay(200)
        pltpu.sync_copy(vec_i, pred_hbm)
        vec_f[...] = conf
        pltpu.touch(vec_f)
        pl.delay(200)
        pltpu.sync_copy(vec_f, conf_hbm)
        vec_f[...] = nconf
        pltpu.touch(vec_f)
        pl.delay(200)
        pltpu.sync_copy(vec_f, nconf_hbm)


_vote_call = functools.partial(
    pl.kernel,
    out_type=[
        jax.ShapeDtypeStruct((16,), jnp.int32),
        jax.ShapeDtypeStruct((16,), jnp.float32),
        jax.ShapeDtypeStruct((16,), jnp.float32),
    ],
    mesh=plsc.VectorSubcoreMesh(core_axis_name="c", subcore_axis_name="s",
                                num_cores=2, num_subcores=NSUB),
    compiler_params=pltpu.CompilerParams(needs_layout_passes=False),
    scratch_types=[
        pltpu.VMEM((CHUNK,), jnp.float32),
        pltpu.VMEM((16,), jnp.float32),
        pltpu.VMEM((16,), jnp.int32),
        pltpu.VMEM((NSUB * 16,), jnp.float32),
        pltpu.VMEM((NSUB * 16,), jnp.int32),
        pltpu.VMEM((16,), jnp.int32),
        pltpu.VMEM((16,), jnp.int32),
        pltpu.VMEM_SHARED((NSUB * 16,), jnp.float32),
        pltpu.VMEM_SHARED((NSUB * 16,), jnp.int32),
        pltpu.SemaphoreType.DMA,
    ],
)(_sc_vote)


def kernel(embedding, embedding_collection, labels_int):
    cos = pl.pallas_call(
        _cos_kernel,
        grid=(GRID,),
        in_specs=[
            pl.BlockSpec((1, D), lambda i: (0, 0)),
            pl.BlockSpec((BLK, D), lambda i: (i, 0)),
        ],
        out_specs=pl.BlockSpec((1, 1, BLK), lambda i: (i, 0, 0)),
        out_shape=jax.ShapeDtypeStruct((GRID, 1, BLK), jnp.float32),
    )(embedding, embedding_collection)
    pred, conf, nconf = _vote_call(cos.reshape(N), labels_int)
    return (pred[0], conf[0], nconf[0])
